# Initial kernel scaffold; baseline (speedup 1.0000x reference)
import jax
import jax.numpy as jnp
from jax.experimental import pallas as pl


def kernel(pred_class, pred_bbox, pred_centerness, mesh):
    raise NotImplementedError("write your pallas kernel here")


# speedup vs baseline: 13.6005x; 13.6005x over previous
import numpy as np
import jax
import jax.numpy as jnp
from jax.experimental import pallas as pl

_LEVEL_HW = [(100, 152), (50, 76), (25, 38), (13, 19), (7, 10)]
_LEVEL_SIZES = [h * w for h, w in _LEVEL_HW]
_BOUNDS = np.cumsum([0] + _LEVEL_SIZES)
_N = int(_BOUNDS[-1])
_NUM_CLASS = 80
_NMS_PRE = 1000
_SCORE_THR = 0.05
_IOU_THR = 0.5
_MAX_PER_IMG = 100
_IMG_H, _IMG_W = 800.0, 1216.0
_NEG = -1e9

_R, _C = 160, 128
_NPAD = _R * _C
_IMIN = np.int32(-2**31)


def _fcos_kernel(pc_ref, bb_ref, ct_ref, mesh_ref, out_ref):
    f32 = jnp.float32

    def cls_body(c, carry):
        m, lab = carry
        v = pc_ref[c]
        upd = v > m
        return jnp.where(upd, v, m), jnp.where(upd, c.astype(f32), lab)

    m0 = pc_ref[0]
    lab0 = jnp.zeros((_R, _C), f32)
    m, lab = jax.lax.fori_loop(1, _NUM_CLASS, cls_body, (m0, lab0))

    p = (jax.lax.broadcasted_iota(jnp.int32, (_R, _C), 0) * _C
         + jax.lax.broadcasted_iota(jnp.int32, (_R, _C), 1))
    valid = p < _N

    mbits = jax.lax.bitcast_convert_type(m, jnp.int32)
    skey = mbits ^ ((mbits >> 31) & jnp.int32(0x7FFFFFFF))

    def cnt(mask):
        return jnp.sum(mask.astype(jnp.int32))

    def topk_mask(lmask, k):
        def bs_body(b, cand):
            bit = jnp.left_shift(jnp.int32(1), jnp.int32(31) - b)
            trial = cand | bit
            trial_s = trial ^ _IMIN
            c = cnt(lmask & (skey >= trial_s))
            return jnp.where(c >= k, trial, cand)

        cand = jax.lax.fori_loop(0, 32, bs_body, jnp.int32(0))
        t_s = cand ^ _IMIN
        gt = lmask & (skey > t_s)
        eq = lmask & (skey == t_s)
        need = jnp.int32(k) - cnt(gt)

        def idx_body(_, lohi):
            lo, hi = lohi
            mid = (lo + hi) // 2
            ok = cnt(eq & (p <= mid)) >= need
            return (jnp.where(ok, lo, mid + 1), jnp.where(ok, mid, hi))

        lo, _ = jax.lax.fori_loop(
            0, 15, idx_body, (jnp.int32(0), jnp.int32(_NPAD - 1)))
        return gt | (eq & (p <= lo))

    sel0 = topk_mask(p < int(_BOUNDS[1]), _NMS_PRE)
    sel1 = topk_mask((p >= int(_BOUNDS[1])) & (p < int(_BOUNDS[2])), _NMS_PRE)
    sel_rest = valid & (p >= int(_BOUNDS[2]))
    sel = sel0 | sel1 | sel_rest

    def sig(x):
        return 1.0 / (1.0 + jnp.exp(-x))

    sc = sig(m) * sig(ct_ref[...])
    sc = jnp.where(sc > _SCORE_THR, sc, f32(_NEG))
    s = jnp.where(sel, sc, f32(_NEG))

    mx, my = mesh_ref[0], mesh_ref[1]
    x1 = jnp.clip(mx - bb_ref[0], 0.0, _IMG_W)
    y1 = jnp.clip(my - bb_ref[1], 0.0, _IMG_H)
    x2 = jnp.clip(mx + bb_ref[2], 0.0, _IMG_W)
    y2 = jnp.clip(my + bb_ref[3], 0.0, _IMG_H)
    area = (x2 - x1) * (y2 - y1)

    lane = jax.lax.broadcasted_iota(jnp.int32, (1, _C), 1)

    def nms_body(i, s):
        bscore = jnp.max(s)
        bidx = jnp.min(jnp.where(s == bscore, p, jnp.int32(_NPAD)))
        pick = (p == bidx).astype(f32)
        bx1 = jnp.sum(pick * x1)
        by1 = jnp.sum(pick * y1)
        bx2 = jnp.sum(pick * x2)
        by2 = jnp.sum(pick * y2)
        blab = jnp.sum(pick * lab)
        barea = jnp.sum(pick * area)

        ix1 = jnp.maximum(bx1, x1)
        iy1 = jnp.maximum(by1, y1)
        ix2 = jnp.minimum(bx2, x2)
        iy2 = jnp.minimum(by2, y2)
        inter = jnp.maximum(ix2 - ix1, 0.0) * jnp.maximum(iy2 - iy1, 0.0)
        iou = inter / (barea + area - inter + 1e-6)
        sup = (iou > _IOU_THR) & (lab == blab)

        row = (jnp.where(lane == 0, bx1, 0.0)
               + jnp.where(lane == 1, by1, 0.0)
               + jnp.where(lane == 2, bx2, 0.0)
               + jnp.where(lane == 3, by2, 0.0)
               + jnp.where(lane == 4, bscore, 0.0)
               + jnp.where(lane == 5, blab, 0.0))
        row = jnp.where(bscore > 0.0, row, 0.0)
        out_ref[pl.ds(i, 1), :] = row

        return jnp.where(sup | (p == bidx), f32(_NEG), s)

    jax.lax.fori_loop(0, _MAX_PER_IMG, nms_body, s)


def _prep(pred_class, pred_bbox, pred_centerness, mesh):
    padn = _NPAD - _N
    pc3 = jnp.pad(pred_class, ((0, padn), (0, 0)),
                  constant_values=-1e30).T.reshape(_NUM_CLASS, _R, _C)
    bb3 = jnp.pad(pred_bbox, ((0, padn), (0, 0))).T.reshape(4, _R, _C)
    ct2 = jnp.pad(pred_centerness, (0, padn)).reshape(_R, _C)
    mesh3 = jnp.pad(mesh, ((0, padn), (0, 0))).T.reshape(2, _R, _C)
    return pc3, bb3, ct2, mesh3


def _run(pc3, bb3, ct2, mesh3, *, interpret=False):
    return pl.pallas_call(
        _fcos_kernel,
        out_shape=jax.ShapeDtypeStruct((_MAX_PER_IMG, _C), jnp.float32),
        interpret=interpret,
    )(pc3, bb3, ct2, mesh3)


def kernel(pred_class, pred_bbox, pred_centerness, mesh):
    out = _run(*_prep(pred_class, pred_bbox, pred_centerness, mesh))
    return out[:, :6]
